# Initial kernel scaffold; baseline (speedup 1.0000x reference)
import jax
import jax.numpy as jnp
from jax.experimental import pallas as pl


def kernel(x, edge_index, edge_attr, Wg1, bg1, Wg2, bg2, Wg3, bg3, gamma, beta, W1, b1, W2, b2):
    raise NotImplementedError("write your pallas kernel here")


# speedup vs baseline: 15.3210x; 15.3210x over previous
import jax
import jax.numpy as jnp
from jax import lax
from jax.experimental import pallas as pl

_N = 100
_NP = 128
_E = 6400
_F32 = jnp.float32
_PH = lax.Precision.HIGHEST


def _dense_body(src_ref, dst_ref, w_ref, xv_ref, wg1_ref, bg1_ref,
                wg2_ref, bg2_ref, wg3_ref, bg3_ref, gam_ref, bet_ref,
                w1_ref, b1_ref, w2g_ref, b2_ref, out_ref):
    col = lax.broadcasted_iota(jnp.int32, (_E, _NP), 1)
    s_oh = (src_ref[...] == col).astype(_F32)
    d_oh = (dst_ref[...] == col).astype(_F32)
    at = lax.dot_general(d_oh, w_ref[...] * s_oh,
                         (((0,), (0,)), ((), ())),
                         precision=_PH, preferred_element_type=_F32)
    deg = jnp.sum(at, axis=1, keepdims=True) + 1.0
    dinv = lax.rsqrt(deg)

    def gcn(h, w, b):
        z = jnp.dot(h, w, preferred_element_type=_F32)
        zh = dinv * z
        agg = jnp.dot(at, zh, precision=_PH, preferred_element_type=_F32) + zh
        return jax.nn.relu(dinv * agg + b)

    h = gcn(xv_ref[...], wg1_ref[...], bg1_ref[...])
    h = gcn(h, wg2_ref[...], bg2_ref[...])
    h = gcn(h, wg3_ref[...], bg3_ref[...])

    rmask = (lax.broadcasted_iota(jnp.int32, (_NP, 1), 0) < _N).astype(_F32)
    mean = jnp.sum(h * rmask, axis=0, keepdims=True) * (1.0 / _N)
    diff = h - mean
    var = jnp.sum(diff * diff * rmask, axis=0, keepdims=True) * (1.0 / _N)
    hn = diff * lax.rsqrt(var + 1e-5) * gam_ref[...] + bet_ref[...]

    l = jax.nn.relu(jnp.dot(hn, w1_ref[...],
                            preferred_element_type=_F32) + b1_ref[...])
    g = lax.dot_general(l, w2g_ref[...], (((0,), (0,)), ((), ())),
                        preferred_element_type=_F32)
    acc = b2_ref[...]
    for c in range(10):
        acc = acc + g[c:c + 1, c * 128:(c + 1) * 128]
    out_ref[...] = acc


def kernel(x, edge_index, edge_attr, Wg1, bg1, Wg2, bg2, Wg3, bg3,
           gamma, beta, W1, b1, W2, b2):
    src = edge_index[0, 0].astype(jnp.int32).reshape(_E, 1)
    dst = edge_index[0, 1].astype(jnp.int32).reshape(_E, 1)
    w = edge_attr[0].reshape(_E, 1)
    xv = jnp.pad(x[0].reshape(_N, 128), ((0, _NP - _N), (0, 0)))
    w2g = jnp.pad(W2.reshape(_N, 1280), ((0, _NP - _N), (0, 0)))
    out = pl.pallas_call(
        _dense_body,
        out_shape=jax.ShapeDtypeStruct((1, 128), _F32),
    )(src, dst, w, xv,
      Wg1, bg1.reshape(1, 64), Wg2, bg2.reshape(1, 128),
      Wg3, bg3.reshape(1, 256), gamma.reshape(1, 256), beta.reshape(1, 256),
      W1, b1.reshape(1, 10), w2g, b2.reshape(1, 128))
    return out.reshape(128)
